# Initial kernel scaffold; baseline (speedup 1.0000x reference)
import jax
import jax.numpy as jnp
from jax.experimental import pallas as pl


def kernel(theta):
    raise NotImplementedError("write your pallas kernel here")


# speedup vs baseline: 2.4001x; 2.4001x over previous
import functools

import jax
import jax.numpy as jnp
from jax import lax
from jax.experimental import pallas as pl
from jax.experimental.pallas import tpu as pltpu
from jax.experimental.pallas import tpu_sc as plsc

N = 128
D = 256
L = 16
NCHUNK = D // L
NC, NS = 2, 16
NW = NC * NS
CPW = N // NW

_mesh = plsc.VectorSubcoreMesh(core_axis_name="c", subcore_axis_name="s")


@functools.partial(
    pl.kernel,
    out_type=(
        jax.ShapeDtypeStruct((N, D), jnp.float32),
        jax.ShapeDtypeStruct((N, D), jnp.float32),
        jax.ShapeDtypeStruct((N, L), jnp.float32),
    ),
    mesh=_mesh,
    compiler_params=pltpu.CompilerParams(needs_layout_passes=False),
    scratch_types=[
        pltpu.VMEM((D, D), jnp.float32),
        pltpu.VMEM((D * L,), jnp.float32),
        pltpu.VMEM((D,), jnp.float32),
        pltpu.VMEM((D,), jnp.float32),
        pltpu.VMEM((L,), jnp.float32),
        pltpu.SemaphoreType.DMA,
    ],
)
def _sums(theta_hbm, v_hbm, r_hbm, sp_hbm, tbuf, pbuf, vbuf, rbuf, sbuf, sem):
    wid = lax.axis_index("s") * NC + lax.axis_index("c")
    zero = jnp.zeros((L,), jnp.float32)
    for k in range(CPW):
        i = wid * CPW + k
        pltpu.async_copy(theta_hbm.at[i], tbuf, sem).wait()

        def rowstep(a, caccs):
            xs = [tbuf[a, pl.ds(c16 * L, L)] for c16 in range(NCHUNK)]
            t = xs
            while len(t) > 1:
                t = [t[j] + t[j + 1] for j in range(0, len(t) - 1, 2)] + (
                    [t[-1]] if len(t) % 2 else [])
            pbuf[pl.ds(a * L, L)] = t[0]
            return tuple(caccs[c16] + xs[c16] for c16 in range(NCHUNK))

        caccs = lax.fori_loop(0, D, rowstep, (zero,) * NCHUNK)
        sacc = zero
        for c16 in range(NCHUNK):
            vbuf[pl.ds(c16 * L, L)] = caccs[c16]
            sacc = sacc + caccs[c16]
        sbuf[...] = sacc
        for g in range(NCHUNK):
            base = jnp.arange(L, dtype=jnp.int32) * L + (g * L * L)
            acc = zero
            for c in range(L):
                acc = acc + plsc.load_gather(pbuf, [base + c])
            rbuf[pl.ds(g * L, L)] = acc
        pltpu.sync_copy(vbuf, v_hbm.at[i])
        pltpu.sync_copy(rbuf, r_hbm.at[i])
        pltpu.sync_copy(sbuf, sp_hbm.at[i])


@functools.partial(
    pl.kernel,
    out_type=jax.ShapeDtypeStruct((N, D, D), jnp.float32),
    mesh=_mesh,
    compiler_params=pltpu.CompilerParams(needs_layout_passes=False),
    scratch_types=[
        pltpu.VMEM((D, D), jnp.float32),
        pltpu.VMEM((N, L), jnp.float32),
        pltpu.VMEM((D,), jnp.float32),
        pltpu.VMEM((D,), jnp.float32),
        pltpu.VMEM((D,), jnp.float32),
        pltpu.VMEM((D + L,), jnp.float32),
        pltpu.VMEM((D,), jnp.float32),
        pltpu.SMEM((N,), jnp.float32),
        pltpu.SMEM((N,), jnp.float32),
        pltpu.SMEM((N,), jnp.float32),
        pltpu.SemaphoreType.DMA,
    ],
)
def _apply(theta_hbm, v_hbm, r_hbm, sp_hbm, out_hbm,
           tbuf, spbuf, vprev, vcur, rnxt, fbuf, gbuf,
           s_sm, c_sm, d_sm, sem):
    wid = lax.axis_index("s") * NC + lax.axis_index("c")
    zero = jnp.zeros((L,), jnp.float32)
    pltpu.sync_copy(sp_hbm, spbuf)

    def cstep(i, c):
        c_sm[i] = c
        s = jnp.sum(spbuf[i])
        s_sm[i] = s
        return s + jnp.float32(D) * c

    lax.fori_loop(0, N, cstep, jnp.float32(0.0))

    d_sm[N - 1] = jnp.float32(0.0)

    def dstep(j, dd):
        i = N - 2 - j
        ddi = s_sm[i + 1] + jnp.float32(D - 1) * c_sm[i + 1] + jnp.float32(D) * dd
        d_sm[i] = ddi
        return ddi

    lax.fori_loop(0, N - 1, dstep, jnp.float32(0.0))

    for k in range(CPW):
        i = wid * CPW + k
        ip = jnp.maximum(i - 1, 0)
        inx = jnp.minimum(i + 1, N - 1)
        pltpu.async_copy(theta_hbm.at[i], tbuf, sem).wait()
        pltpu.sync_copy(v_hbm.at[ip], vprev)
        pltpu.sync_copy(v_hbm.at[i], vcur)
        pltpu.sync_copy(r_hbm.at[inx], rnxt)
        cprev = c_sm[ip]
        ccur = c_sm[i]
        coef = jnp.float32(D - 1) * ccur + d_sm[inx]

        @pl.when(i > 0)
        def _():
            for c16 in range(NCHUNK):
                sl = pl.ds(c16 * L, L)
                fbuf[sl] = vprev[sl] + cprev

        @pl.when(i == 0)
        def _():
            for c16 in range(NCHUNK):
                fbuf[pl.ds(c16 * L, L)] = zero

        @pl.when(i < N - 1)
        def _():
            for c16 in range(NCHUNK):
                sl = pl.ds(c16 * L, L)
                gbuf[sl] = rnxt[sl] + jnp.float32(D - 1) * vcur[sl] + coef

        @pl.when(i == N - 1)
        def _():
            for c16 in range(NCHUNK):
                gbuf[pl.ds(c16 * L, L)] = zero

        gregs = tuple(gbuf[pl.ds(c16 * L, L)] for c16 in range(NCHUNK))

        def rowstep(a, gs):
            fs = fbuf[pl.ds(a, L)][0]
            for c16 in range(NCHUNK):
                sl = pl.ds(c16 * L, L)
                tbuf[a, sl] = tbuf[a, sl] + gs[c16] + fs
            return gs

        lax.fori_loop(0, D, rowstep, gregs)
        pltpu.sync_copy(tbuf, out_hbm.at[i])


def kernel(theta):
    v, r, sp = _sums(theta)
    return _apply(theta, v, r, sp)
